# baseline (speedup 1.0000x reference)
import functools

import jax
import jax.numpy as jnp
from jax import lax
from jax.experimental import pallas as pl
from jax.experimental.pallas import tpu as pltpu
from jax.experimental.pallas import tpu_sc as plsc

_NUM_CORES = 2
_NUM_SUBCORES = 16
_NUM_WORKERS = _NUM_CORES * _NUM_SUBCORES

_TC_ROWS = 1024
_TC_BLOCK_ROWS = 512


def kernel(tokens, W_pos):
    batch, seq_len = tokens.shape
    n_ctx, d_model = W_pos.shape

    sc_rows = seq_len - _TC_ROWS
    rows_per_w = sc_rows // _NUM_WORKERS

    sc_mesh = plsc.VectorSubcoreMesh(
        core_axis_name="c",
        subcore_axis_name="s",
        num_cores=_NUM_CORES,
        num_subcores=_NUM_SUBCORES,
    )

    @functools.partial(
        pl.kernel,
        out_type=pltpu.HBM((batch, seq_len, d_model), W_pos.dtype),
        mesh=sc_mesh,
        scratch_types=[
            pltpu.VMEM((rows_per_w, d_model), W_pos.dtype),
            pltpu.SemaphoreType.DMA,
        ],
    )
    def sc_fill(w_hbm, out_hbm, buf, sem):
        wid = lax.axis_index("s") * _NUM_CORES + lax.axis_index("c")
        base = _TC_ROWS + wid * rows_per_w
        pltpu.sync_copy(w_hbm.at[pl.ds(base, rows_per_w), :], buf)
        copies = [
            pltpu.async_copy(buf, out_hbm.at[b, pl.ds(base, rows_per_w), :], sem)
            for b in range(batch)
        ]
        for c in copies:
            c.wait()

    def tc_body(w_ref, alias_ref, out_ref):
        del alias_ref
        out_ref[...] = w_ref[...][None]

    sc_out = sc_fill(W_pos)

    n_blocks = _TC_ROWS // _TC_BLOCK_ROWS
    return pl.pallas_call(
        tc_body,
        grid=(n_blocks, batch),
        in_specs=[
            pl.BlockSpec((_TC_BLOCK_ROWS, d_model), lambda i, b: (i, 0)),
            pl.BlockSpec(memory_space=pl.ANY),
        ],
        out_specs=pl.BlockSpec(
            (1, _TC_BLOCK_ROWS, d_model), lambda i, b: (b, i, 0)
        ),
        out_shape=jax.ShapeDtypeStruct((batch, seq_len, d_model), W_pos.dtype),
        input_output_aliases={1: 0},
    )(W_pos, sc_out)
